# Initial kernel scaffold; baseline (speedup 1.0000x reference)
import jax
import jax.numpy as jnp
from jax.experimental import pallas as pl


def kernel(x, edge_index, coords, batch, params):
    raise NotImplementedError("write your pallas kernel here")


# speedup vs baseline: 2.5568x; 2.5568x over previous
import functools
import jax
import jax.numpy as jnp
from jax import lax
from jax.experimental import pallas as pl
from jax.experimental.pallas import tpu as pltpu
from jax.experimental.pallas import tpu_sc as plsc

N = 10000
E = 160000
HD = 64
NG = 32

NC = 2
NS = 16
NW = NC * NS
EPW = E // NW
CHUNK = 1000
NCHUNK = EPW // CHUNK
NPW = N // NS

BE = 2000
BN = 2000


@functools.cache
def _make_gather_pair(D):
    mesh = plsc.VectorSubcoreMesh(core_axis_name="c", subcore_axis_name="s", num_cores=NC, num_subcores=NS)

    @functools.partial(
        pl.kernel,
        out_type=(jax.ShapeDtypeStruct((E, D), jnp.float32),
                  jax.ShapeDtypeStruct((E, D), jnp.float32)),
        mesh=mesh,
        scratch_types=[pltpu.VMEM((CHUNK,), jnp.int32),
                       pltpu.VMEM((CHUNK, D), jnp.float32),
                       pltpu.SemaphoreType.DMA],
        compiler_params=pltpu.CompilerParams(use_tc_tiling_on_sc=False),
        name=f"sc_gather_pair_{D}",
    )
    def gather2(table_hbm, row_hbm, col_hbm, outr_hbm, outc_hbm,
                idx_v, rows_v, sem):
        wid = lax.axis_index("s") * NC + lax.axis_index("c")

        def body(c, _):
            base = pl.multiple_of(wid * EPW + c * CHUNK, CHUNK)
            pltpu.sync_copy(row_hbm.at[pl.ds(base, CHUNK)], idx_v)
            pltpu.async_copy(table_hbm.at[idx_v], rows_v, sem).wait()
            pltpu.sync_copy(rows_v, outr_hbm.at[pl.ds(base, CHUNK)])
            pltpu.sync_copy(col_hbm.at[pl.ds(base, CHUNK)], idx_v)
            pltpu.async_copy(table_hbm.at[idx_v], rows_v, sem).wait()
            pltpu.sync_copy(rows_v, outc_hbm.at[pl.ds(base, CHUNK)])
            return _

        lax.fori_loop(0, NCHUNK, body, None)

    return gather2


@functools.cache
def _make_scatter(D):
    mesh = plsc.VectorSubcoreMesh(core_axis_name="c", subcore_axis_name="s", num_cores=NC, num_subcores=NS)

    @functools.partial(
        pl.kernel,
        out_type=jax.ShapeDtypeStruct((NC, N, D), jnp.float32),
        mesh=mesh,
        scratch_types=[pltpu.VMEM((CHUNK,), jnp.int32),
                       pltpu.VMEM((CHUNK, D), jnp.float32),
                       pltpu.VMEM_SHARED((N, D), jnp.float32),
                       pltpu.SemaphoreType.DMA],
        compiler_params=pltpu.CompilerParams(use_tc_tiling_on_sc=False),
        name=f"sc_scatter_add_{D}",
    )
    def scat(vals_hbm, idx_hbm, zeros_hbm, out_hbm,
             idx_v, vals_v, acc_sh, sem):
        cid = lax.axis_index("c")
        sid = lax.axis_index("s")
        wid = sid * NC + cid
        rbase = pl.multiple_of(sid * NPW, NPW)
        pltpu.sync_copy(zeros_hbm.at[pl.ds(rbase, NPW)],
                        acc_sh.at[pl.ds(rbase, NPW)])
        plsc.subcore_barrier()

        def body(c, _):
            base = pl.multiple_of(wid * EPW + c * CHUNK, CHUNK)
            pltpu.sync_copy(idx_hbm.at[pl.ds(base, CHUNK)], idx_v)
            pltpu.sync_copy(vals_hbm.at[pl.ds(base, CHUNK)], vals_v)
            pltpu.sync_copy(vals_v, acc_sh.at[idx_v], add=True)
            return _

        lax.fori_loop(0, NCHUNK, body, None)
        plsc.subcore_barrier()
        pltpu.sync_copy(acc_sh.at[pl.ds(rbase, NPW)],
                        out_hbm.at[cid].at[pl.ds(rbase, NPW)])

    return scat


def _gather64(table, row, col):
    return _make_gather_pair(64)(table, row, col)


def _gather16(table, row, col):
    return _make_gather_pair(16)(table, row, col)


def _scatter16(vals, idx, zeros):
    return _make_scatter(16)(vals, idx, zeros)


def _scatter64(vals, idx, zeros):
    return _make_scatter(64)(vals, idx, zeros)


def _eblk(i):
    return (i, 0)


def _wblk(i):
    return (0, 0)


def _espec(d):
    return pl.BlockSpec((BE, d), _eblk)


def _wspec(shape):
    return pl.BlockSpec(shape, _wblk)


def _prep_kernel(x_ref, w_ref, b_ref, o_ref):
    o_ref[...] = x_ref[...] * w_ref[...] + b_ref[...]


def _prep(x, in_w, in_b):
    return pl.pallas_call(
        _prep_kernel,
        grid=(N // BN,),
        in_specs=[pl.BlockSpec((BN, 1), _eblk), _wspec((1, HD)), _wspec((1, HD))],
        out_specs=pl.BlockSpec((BN, HD), _eblk),
        out_shape=jax.ShapeDtypeStruct((N, HD), jnp.float32),
    )(x, in_w.reshape(1, HD), in_b.reshape(1, HD))


def _silu(v):
    return v * jax.nn.sigmoid(v)


def _coord_mlp_body(two_rel, xr, xc, ra, rb, w1a, w1b, w1c, b1, w2, b2,
                    eww, ewb, scale, delta_o):
    if two_rel:
        rel = ra[...] - rb[...]
    else:
        rel = ra[...]
    h = (jnp.dot(xr[...], w1a[...], preferred_element_type=jnp.float32)
         + jnp.dot(xc[...], w1b[...], preferred_element_type=jnp.float32)
         + jnp.dot(rel, w1c[...], preferred_element_type=jnp.float32)
         + b1[...])
    h = _silu(h)
    d = jnp.dot(h, w2[...], preferred_element_type=jnp.float32) + b2[...]
    nrm = jnp.sqrt(jnp.sum(d * d, axis=-1, keepdims=True))
    nrm = jnp.maximum(nrm, 1e-8)
    d = d / nrm * scale[...]
    ew = jax.nn.sigmoid(
        jnp.sum(rel * eww[...], axis=-1, keepdims=True) + ewb[...])
    delta_o[...] = d * ew


def _make_coord_mlp(two_rel):
    n_rel = 2 if two_rel else 1

    def body(*refs):
        if two_rel:
            _coord_mlp_body(True, *refs)
        else:
            _coord_mlp_body(False, refs[0], refs[1], refs[2], None, *refs[3:])

    return pl.pallas_call(
        body,
        grid=(E // BE,),
        in_specs=[_espec(HD), _espec(HD)] + [_espec(16)] * n_rel + [
            _wspec((HD, 2 * HD)), _wspec((HD, 2 * HD)), _wspec((16, 2 * HD)),
            _wspec((1, 2 * HD)), _wspec((2 * HD, 16)), _wspec((1, 16)),
            _wspec((1, 16)), _wspec((1, 1)), _wspec((1, 1))],
        out_specs=_espec(16),
        out_shape=jax.ShapeDtypeStruct((E, 16), jnp.float32),
    )


_coord_mlp_l0 = _make_coord_mlp(True)
_coord_mlp_ln = _make_coord_mlp(False)


def _edge_mlp_body(two_rel, xr, xc, ra, rb, dcr, dcc, w1a, w1b, w1c, b1,
                   w2, b2, e_o, rel_o):
    if two_rel:
        rel = ra[...] - rb[...] + dcr[...] - dcc[...]
    else:
        rel = ra[...] + dcr[...] - dcc[...]
    h = (jnp.dot(xr[...], w1a[...], preferred_element_type=jnp.float32)
         + jnp.dot(xc[...], w1b[...], preferred_element_type=jnp.float32)
         + jnp.dot(rel, w1c[...], preferred_element_type=jnp.float32)
         + b1[...])
    h = _silu(h)
    e_o[...] = _silu(
        jnp.dot(h, w2[...], preferred_element_type=jnp.float32) + b2[...])
    rel_o[...] = rel


def _make_edge_mlp(two_rel):
    n_rel = 2 if two_rel else 1

    def body(*refs):
        if two_rel:
            _edge_mlp_body(True, *refs)
        else:
            xr, xc, ra, dcr, dcc = refs[:5]
            _edge_mlp_body(False, xr, xc, ra, None, dcr, dcc, *refs[5:])

    return pl.pallas_call(
        body,
        grid=(E // BE,),
        in_specs=[_espec(HD), _espec(HD)] + [_espec(16)] * (n_rel + 2) + [
            _wspec((HD, 2 * HD)), _wspec((HD, 2 * HD)), _wspec((16, 2 * HD)),
            _wspec((1, 2 * HD)), _wspec((2 * HD, HD)), _wspec((1, HD))],
        out_specs=(_espec(HD), _espec(16)),
        out_shape=(jax.ShapeDtypeStruct((E, HD), jnp.float32),
                   jax.ShapeDtypeStruct((E, 16), jnp.float32)),
    )


_edge_mlp_l0 = _make_edge_mlp(True)
_edge_mlp_ln = _make_edge_mlp(False)


def _node_mlp_body(x, a0, a1, w1a, w1b, b1, w2, b2, o):
    xv = x[...]
    agg = a0[...] + a1[...]
    h = _silu(jnp.dot(xv, w1a[...], preferred_element_type=jnp.float32)
              + jnp.dot(agg, w1b[...], preferred_element_type=jnp.float32)
              + b1[...])
    xn = xv + jnp.dot(h, w2[...], preferred_element_type=jnp.float32) + b2[...]
    mu = jnp.mean(xn, axis=-1, keepdims=True)
    var = jnp.mean((xn - mu) ** 2, axis=-1, keepdims=True)
    o[...] = (xn - mu) / jnp.sqrt(var + 1e-5)


def _node_mlp(x, a0, a1, w1a, w1b, b1, w2, b2):
    nspec = pl.BlockSpec((BN, HD), _eblk)
    return pl.pallas_call(
        _node_mlp_body,
        grid=(N // BN,),
        in_specs=[nspec, nspec, nspec,
                  _wspec((HD, 2 * HD)), _wspec((HD, 2 * HD)),
                  _wspec((1, 2 * HD)), _wspec((2 * HD, HD)), _wspec((1, HD))],
        out_specs=nspec,
        out_shape=jax.ShapeDtypeStruct((N, HD), jnp.float32),
    )(x, a0, a1, w1a, w1b, b1, w2, b2)


def _pair_add_body(p_ref, o_ref):
    o_ref[...] = p_ref[0] + p_ref[1]


def _pair_add(p):
    return pl.pallas_call(
        _pair_add_body,
        grid=(N // BN,),
        in_specs=[pl.BlockSpec((2, BN, 16), lambda i: (0, i, 0))],
        out_specs=pl.BlockSpec((BN, 16), _eblk),
        out_shape=jax.ShapeDtypeStruct((N, 16), jnp.float32),
    )(p)


def _head_body(x, batch, w, b, o):
    onehot = (batch[...] == lax.broadcasted_iota(jnp.int32, (1, NG), 1))
    onehot = onehot.astype(jnp.float32)
    sums = lax.dot_general(onehot, x[...], (((0,), (0,)), ((), ())),
                           preferred_element_type=jnp.float32)
    ones = jnp.ones((N, 1), jnp.float32)
    cnt = lax.dot_general(onehot, ones, (((0,), (0,)), ((), ())),
                          preferred_element_type=jnp.float32)
    pooled = sums / jnp.maximum(cnt, 1.0)
    out = (jnp.dot(jax.nn.relu(pooled), w[...],
                   preferred_element_type=jnp.float32) + b[...])
    nrm = jnp.sqrt(jnp.sum(out * out, axis=-1, keepdims=True))
    o[...] = out / jnp.maximum(nrm, 1e-12)


def _head(x, batch2d, out_w, out_b):
    return pl.pallas_call(
        _head_body,
        grid=(1,),
        in_specs=[pl.BlockSpec((N, HD), _eblk), pl.BlockSpec((N, 1), _eblk),
                  _wspec((HD, 2 * HD)), _wspec((1, 2 * HD))],
        out_specs=pl.BlockSpec((NG, 2 * HD), _eblk),
        out_shape=jax.ShapeDtypeStruct((NG, 2 * HD), jnp.float32),
    )(x, batch2d, out_w, out_b)


def _pad16(w):
    return jnp.pad(w, ((0, 16 - w.shape[0]), (0, 0)))


def kernel(x, edge_index, coords, batch, params):
    row = edge_index[0]
    col = edge_index[1]
    cpad = jnp.pad(coords, ((0, 0), (0, 13)))
    zeros16 = jnp.zeros((N, 16), jnp.float32)
    zeros64 = jnp.zeros((N, HD), jnp.float32)

    h = _prep(x, params["in_W"], params["in_b"])
    cr, cc = _gather16(cpad, row, col)
    rel = None

    for li, p in enumerate(params["layers"]):
        xr, xc = _gather64(h, row, col)
        cw1a = p["c_W1"][:HD]
        cw1b = p["c_W1"][HD:2 * HD]
        cw1c = _pad16(p["c_W1"][2 * HD:])
        cw2 = jnp.pad(p["c_W2"], ((0, 0), (0, 13)))
        cb2 = jnp.pad(p["c_b2"], (0, 13)).reshape(1, 16)
        eww = _pad16(p["ew_W"]).reshape(1, 16)
        ewb = p["ew_b"].reshape(1, 1)
        scale = p["scale"].reshape(1, 1)
        common = (cw1a, cw1b, cw1c, p["c_b1"].reshape(1, 2 * HD), cw2, cb2,
                  eww, ewb, scale)
        if li == 0:
            delta = _coord_mlp_l0(xr, xc, cr, cc, *common)
        else:
            delta = _coord_mlp_ln(xr, xc, rel, *common)

        dc = _scatter16(delta, row, zeros16)
        dcsum = _pair_add(dc)
        dcr, dcc = _gather16(dcsum, row, col)

        ew1a = p["e_W1"][:HD]
        ew1b = p["e_W1"][HD:2 * HD]
        ew1c = _pad16(p["e_W1"][2 * HD:])
        ecommon = (ew1a, ew1b, ew1c, p["e_b1"].reshape(1, 2 * HD),
                   p["e_W2"], p["e_b2"].reshape(1, HD))
        if li == 0:
            e_ij, rel = _edge_mlp_l0(xr, xc, cr, cc, dcr, dcc, *ecommon)
        else:
            e_ij, rel = _edge_mlp_ln(xr, xc, rel, dcr, dcc, *ecommon)

        agg = _scatter64(e_ij, row, zeros64)
        h = _node_mlp(h, agg[0], agg[1],
                      p["n_W1"][:HD], p["n_W1"][HD:],
                      p["n_b1"].reshape(1, 2 * HD),
                      p["n_W2"], p["n_b2"].reshape(1, HD))

    return _head(h, batch.reshape(N, 1), params["out_W"],
                 params["out_b"].reshape(1, 2 * HD))


# speedup vs baseline: 2.9187x; 1.1415x over previous
import functools
import jax
import jax.numpy as jnp
from jax import lax
from jax.experimental import pallas as pl
from jax.experimental.pallas import tpu as pltpu
from jax.experimental.pallas import tpu_sc as plsc

N = 10000
E = 160000
HD = 64
NG = 32
W128 = 128

NC = 2
NS = 16
NW = NC * NS
EPW = E // NW
NPW = N // NS

BE = 2000
BN = 2000


def _make_gather_pair128():
    mesh = plsc.VectorSubcoreMesh(core_axis_name="c", subcore_axis_name="s",
                                  num_cores=NC, num_subcores=NS)
    CH = 200
    NCH = EPW // CH

    @functools.partial(
        pl.kernel,
        out_type=(jax.ShapeDtypeStruct((E, W128), jnp.float32),
                  jax.ShapeDtypeStruct((E, W128), jnp.float32)),
        mesh=mesh,
        scratch_types=[pltpu.VMEM((CH,), jnp.int32),
                       pltpu.VMEM((CH, W128), jnp.float32),
                       pltpu.SemaphoreType.DMA],
        name="sc_gather_pair_128",
    )
    def gather2(table_hbm, row_hbm, col_hbm, outr_hbm, outc_hbm,
                idx_v, rows_v, sem):
        wid = lax.axis_index("s") * NC + lax.axis_index("c")

        def body(c, _):
            base = pl.multiple_of(wid * EPW + c * CH, CH)
            pltpu.sync_copy(row_hbm.at[pl.ds(base, CH)], idx_v)
            pltpu.async_copy(table_hbm.at[idx_v], rows_v, sem).wait()
            pltpu.sync_copy(rows_v, outr_hbm.at[pl.ds(base, CH)])
            pltpu.sync_copy(col_hbm.at[pl.ds(base, CH)], idx_v)
            pltpu.async_copy(table_hbm.at[idx_v], rows_v, sem).wait()
            pltpu.sync_copy(rows_v, outc_hbm.at[pl.ds(base, CH)])
            return _

        lax.fori_loop(0, NCH, body, None)

    return gather2


def _make_scatter128():
    mesh = plsc.VectorSubcoreMesh(core_axis_name="c", subcore_axis_name="s",
                                  num_cores=NC, num_subcores=NS)
    CH = 200
    NCH = EPW // CH
    SW = 632
    SL = N - 15 * SW

    @functools.partial(
        pl.kernel,
        out_type=jax.ShapeDtypeStruct((NC, N, W128), jnp.float32),
        mesh=mesh,
        scratch_types=[pltpu.VMEM((CH,), jnp.int32),
                       pltpu.VMEM((CH, W128), jnp.float32),
                       pltpu.VMEM_SHARED((N, W128), jnp.float32),
                       pltpu.SemaphoreType.DMA],
        name="sc_scatter_add_128",
    )
    def scat(vals_hbm, idx_hbm, zeros_hbm, out_hbm,
             idx_v, vals_v, acc_sh, sem):
        cid = lax.axis_index("c")
        sid = lax.axis_index("s")
        wid = sid * NC + cid
        rbase = pl.multiple_of(sid * SW, 8)

        @pl.when(sid < 15)
        def _():
            pltpu.sync_copy(zeros_hbm.at[pl.ds(rbase, SW)],
                            acc_sh.at[pl.ds(rbase, SW)])

        @pl.when(sid == 15)
        def _():
            pltpu.sync_copy(zeros_hbm.at[pl.ds(15 * SW, SL)],
                            acc_sh.at[pl.ds(15 * SW, SL)])

        plsc.subcore_barrier()

        def body(c, _):
            base = pl.multiple_of(wid * EPW + c * CH, CH)
            pltpu.sync_copy(idx_hbm.at[pl.ds(base, CH)], idx_v)
            pltpu.sync_copy(vals_hbm.at[pl.ds(base, CH)], vals_v)
            pltpu.sync_copy(vals_v, acc_sh.at[idx_v], add=True)
            return _

        lax.fori_loop(0, NCH, body, None)
        plsc.subcore_barrier()

        @pl.when(sid < 15)
        def _():
            pltpu.sync_copy(acc_sh.at[pl.ds(rbase, SW)],
                            out_hbm.at[cid].at[pl.ds(rbase, SW)])

        @pl.when(sid == 15)
        def _():
            pltpu.sync_copy(acc_sh.at[pl.ds(15 * SW, SL)],
                            out_hbm.at[cid].at[pl.ds(15 * SW, SL)])

    return scat


def _make_scatter16():
    mesh = plsc.VectorSubcoreMesh(core_axis_name="c", subcore_axis_name="s",
                                  num_cores=1, num_subcores=NS)
    CH = 1000
    epw = E // NS
    NCH = epw // CH

    @functools.partial(
        pl.kernel,
        out_type=jax.ShapeDtypeStruct((N, 16), jnp.float32),
        mesh=mesh,
        scratch_types=[pltpu.VMEM((CH,), jnp.int32),
                       pltpu.VMEM((CH, 16), jnp.float32),
                       pltpu.VMEM_SHARED((N, 16), jnp.float32),
                       pltpu.SemaphoreType.DMA],
        compiler_params=pltpu.CompilerParams(use_tc_tiling_on_sc=False),
        name="sc_scatter_add_16",
    )
    def scat(vals_hbm, idx_hbm, zeros_hbm, out_hbm,
             idx_v, vals_v, acc_sh, sem):
        sid = lax.axis_index("s")
        rbase = pl.multiple_of(sid * NPW, NPW)
        pltpu.sync_copy(zeros_hbm.at[pl.ds(rbase, NPW)],
                        acc_sh.at[pl.ds(rbase, NPW)])
        plsc.subcore_barrier()

        def body(c, _):
            base = pl.multiple_of(sid * epw + c * CH, CH)
            pltpu.sync_copy(idx_hbm.at[pl.ds(base, CH)], idx_v)
            pltpu.sync_copy(vals_hbm.at[pl.ds(base, CH)], vals_v)
            pltpu.sync_copy(vals_v, acc_sh.at[idx_v], add=True)
            return _

        lax.fori_loop(0, NCH, body, None)
        plsc.subcore_barrier()
        pltpu.sync_copy(acc_sh.at[pl.ds(rbase, NPW)],
                        out_hbm.at[pl.ds(rbase, NPW)])

    return scat


def _make_gather_sub16():
    mesh = plsc.VectorSubcoreMesh(core_axis_name="c", subcore_axis_name="s",
                                  num_cores=NC, num_subcores=NS)
    CH = 1000
    NCH = EPW // CH

    @functools.partial(
        pl.kernel,
        out_type=jax.ShapeDtypeStruct((E, 16), jnp.float32),
        mesh=mesh,
        scratch_types=[pltpu.VMEM((CH,), jnp.int32),
                       pltpu.VMEM((CH, 16), jnp.float32),
                       pltpu.VMEM((CH, 16), jnp.float32),
                       pltpu.SemaphoreType.DMA],
        compiler_params=pltpu.CompilerParams(use_tc_tiling_on_sc=False),
        name="sc_gather_sub_16",
    )
    def gsub(table_hbm, row_hbm, col_hbm, out_hbm,
             idx_v, ra_v, rb_v, sem):
        wid = lax.axis_index("s") * NC + lax.axis_index("c")

        def body(c, _):
            base = pl.multiple_of(wid * EPW + c * CH, CH)
            pltpu.sync_copy(row_hbm.at[pl.ds(base, CH)], idx_v)
            pltpu.async_copy(table_hbm.at[idx_v], ra_v, sem).wait()
            pltpu.sync_copy(col_hbm.at[pl.ds(base, CH)], idx_v)
            pltpu.async_copy(table_hbm.at[idx_v], rb_v, sem).wait()

            def sub_body(i, _):
                ra_v[i] = ra_v[i] - rb_v[i]
                return _

            lax.fori_loop(0, CH, sub_body, None)
            pltpu.sync_copy(ra_v, out_hbm.at[pl.ds(base, CH)])
            return _

        lax.fori_loop(0, NCH, body, None)

    return gsub


@functools.cache
def _sc_kernels():
    return (_make_gather_pair128(), _make_scatter128(),
            _make_scatter16(), _make_gather_sub16())


def _gather128(table, row, col):
    return _sc_kernels()[0](table, row, col)


def _scatter128(vals, idx, zeros):
    return _sc_kernels()[1](vals, idx, zeros)


def _scatter16(vals, idx, zeros):
    return _sc_kernels()[2](vals, idx, zeros)


def _gather_sub16(table, row, col):
    return _sc_kernels()[3](table, row, col)


def _eblk(i):
    return (i, 0)


def _wblk(i):
    return (0, 0)


def _espec(d):
    return pl.BlockSpec((BE, d), _eblk)


def _wspec(shape):
    return pl.BlockSpec(shape, _wblk)


def _padrows(w):
    return jnp.pad(w, ((0, W128 - w.shape[0]), (0, 0)))


def _silu(v):
    return v * jax.nn.sigmoid(v)


def _prep_kernel(x_ref, w_ref, b_ref, o_ref):
    x0 = x_ref[...] * w_ref[...] + b_ref[...]
    o_ref[...] = jnp.concatenate(
        [x0, jnp.zeros((BN, W128 - HD), jnp.float32)], axis=-1)


def _prep(x, in_w, in_b):
    return pl.pallas_call(
        _prep_kernel,
        grid=(N // BN,),
        in_specs=[pl.BlockSpec((BN, 1), _eblk), _wspec((1, HD)),
                  _wspec((1, HD))],
        out_specs=pl.BlockSpec((BN, W128), _eblk),
        out_shape=jax.ShapeDtypeStruct((N, W128), jnp.float32),
    )(x, in_w.reshape(1, HD), in_b.reshape(1, HD))


def _coord_mlp_body(xr, xc, ra, w1a, w1b, w1c, b1, w2, b2,
                    eww, ewb, scale, delta_o):
    rel = ra[...]
    h = (jnp.dot(xr[...], w1a[...], preferred_element_type=jnp.float32)
         + jnp.dot(xc[...], w1b[...], preferred_element_type=jnp.float32)
         + jnp.dot(rel, w1c[...], preferred_element_type=jnp.float32)
         + b1[...])
    h = _silu(h)
    d = jnp.dot(h, w2[...], preferred_element_type=jnp.float32) + b2[...]
    nrm = jnp.sqrt(jnp.sum(d * d, axis=-1, keepdims=True))
    nrm = jnp.maximum(nrm, 1e-8)
    d = d / nrm * scale[...]
    ew = jax.nn.sigmoid(
        jnp.sum(rel * eww[...], axis=-1, keepdims=True) + ewb[...])
    delta_o[...] = d * ew


_coord_mlp = pl.pallas_call(
    _coord_mlp_body,
    grid=(E // BE,),
    in_specs=[_espec(W128), _espec(W128), _espec(16),
              _wspec((W128, 2 * HD)), _wspec((W128, 2 * HD)),
              _wspec((16, 2 * HD)), _wspec((1, 2 * HD)),
              _wspec((2 * HD, 16)), _wspec((1, 16)),
              _wspec((1, 16)), _wspec((1, 1)), _wspec((1, 1))],
    out_specs=_espec(16),
    out_shape=jax.ShapeDtypeStruct((E, 16), jnp.float32),
)


def _edge_mlp_body(xr, xc, ra, drel, w1a, w1b, w1c, b1, w2, b2, e_o, rel_o):
    rel = ra[...] + drel[...]
    h = (jnp.dot(xr[...], w1a[...], preferred_element_type=jnp.float32)
         + jnp.dot(xc[...], w1b[...], preferred_element_type=jnp.float32)
         + jnp.dot(rel, w1c[...], preferred_element_type=jnp.float32)
         + b1[...])
    h = _silu(h)
    e = _silu(
        jnp.dot(h, w2[...], preferred_element_type=jnp.float32) + b2[...])
    e_o[...] = jnp.concatenate(
        [e, jnp.zeros((BE, W128 - HD), jnp.float32)], axis=-1)
    rel_o[...] = rel


_edge_mlp = pl.pallas_call(
    _edge_mlp_body,
    grid=(E // BE,),
    in_specs=[_espec(W128), _espec(W128), _espec(16), _espec(16),
              _wspec((W128, 2 * HD)), _wspec((W128, 2 * HD)),
              _wspec((16, 2 * HD)), _wspec((1, 2 * HD)),
              _wspec((2 * HD, HD)), _wspec((1, HD))],
    out_specs=(pl.BlockSpec((BE, W128), _eblk), _espec(16)),
    out_shape=(jax.ShapeDtypeStruct((E, W128), jnp.float32),
               jax.ShapeDtypeStruct((E, 16), jnp.float32)),
)


def _node_mlp_body(x, a0, a1, w1a, w1b, b1, w2, b2, o):
    xv = x[...]
    agg = a0[...] + a1[...]
    h = _silu(jnp.dot(xv, w1a[...], preferred_element_type=jnp.float32)
              + jnp.dot(agg, w1b[...], preferred_element_type=jnp.float32)
              + b1[...])
    xn = xv + jnp.dot(h, w2[...], preferred_element_type=jnp.float32) + b2[...]
    mu = jnp.sum(xn, axis=-1, keepdims=True) * (1.0 / HD)
    lane = lax.broadcasted_iota(jnp.int32, (1, W128), 1)
    xc = jnp.where(lane < HD, xn - mu, 0.0)
    var = jnp.sum(xc * xc, axis=-1, keepdims=True) * (1.0 / HD)
    o[...] = xc / jnp.sqrt(var + 1e-5)


def _node_mlp(x, a0, a1, w1a, w1b, b1, w2, b2):
    nspec = pl.BlockSpec((BN, W128), _eblk)
    return pl.pallas_call(
        _node_mlp_body,
        grid=(N // BN,),
        in_specs=[nspec, nspec, nspec,
                  _wspec((W128, 2 * HD)), _wspec((W128, 2 * HD)),
                  _wspec((1, 2 * HD)), _wspec((2 * HD, W128)),
                  _wspec((1, W128))],
        out_specs=nspec,
        out_shape=jax.ShapeDtypeStruct((N, W128), jnp.float32),
    )(x, a0, a1, w1a, w1b, b1, w2, b2)


def _head_body(x, batch, w, b, o):
    onehot = (batch[...] == lax.broadcasted_iota(jnp.int32, (1, NG), 1))
    onehot = onehot.astype(jnp.float32)
    sums = lax.dot_general(onehot, x[...], (((0,), (0,)), ((), ())),
                           preferred_element_type=jnp.float32)
    ones = jnp.ones((N, 1), jnp.float32)
    cnt = lax.dot_general(onehot, ones, (((0,), (0,)), ((), ())),
                          preferred_element_type=jnp.float32)
    pooled = sums / jnp.maximum(cnt, 1.0)
    out = (jnp.dot(jax.nn.relu(pooled), w[...],
                   preferred_element_type=jnp.float32) + b[...])
    nrm = jnp.sqrt(jnp.sum(out * out, axis=-1, keepdims=True))
    o[...] = out / jnp.maximum(nrm, 1e-12)


def _head(x, batch2d, out_w, out_b):
    return pl.pallas_call(
        _head_body,
        grid=(1,),
        in_specs=[pl.BlockSpec((N, W128), _eblk), pl.BlockSpec((N, 1), _eblk),
                  _wspec((W128, 2 * HD)), _wspec((1, 2 * HD))],
        out_specs=pl.BlockSpec((NG, 2 * HD), _eblk),
        out_shape=jax.ShapeDtypeStruct((NG, 2 * HD), jnp.float32),
    )(x, batch2d, out_w, out_b)


def _pad16(w):
    return jnp.pad(w, ((0, 16 - w.shape[0]), (0, 0)))


def kernel(x, edge_index, coords, batch, params):
    row = edge_index[0]
    col = edge_index[1]
    cpad = jnp.pad(coords, ((0, 0), (0, 13)))
    zeros16 = jnp.zeros((N, 16), jnp.float32)
    zeros128 = jnp.zeros((N, W128), jnp.float32)

    h = _prep(x, params["in_W"], params["in_b"])
    rel = _gather_sub16(cpad, row, col)

    for p in params["layers"]:
        xr, xc = _gather128(h, row, col)
        cw1a = _padrows(p["c_W1"][:HD])
        cw1b = _padrows(p["c_W1"][HD:2 * HD])
        cw1c = _pad16(p["c_W1"][2 * HD:])
        cw2 = jnp.pad(p["c_W2"], ((0, 0), (0, 13)))
        cb2 = jnp.pad(p["c_b2"], (0, 13)).reshape(1, 16)
        eww = _pad16(p["ew_W"]).reshape(1, 16)
        delta = _coord_mlp(xr, xc, rel,
                           cw1a, cw1b, cw1c, p["c_b1"].reshape(1, 2 * HD),
                           cw2, cb2, eww, p["ew_b"].reshape(1, 1),
                           p["scale"].reshape(1, 1))

        dc = _scatter16(delta, row, zeros16)
        drel = _gather_sub16(dc, row, col)

        e128, rel = _edge_mlp(xr, xc, rel, drel,
                              _padrows(p["e_W1"][:HD]),
                              _padrows(p["e_W1"][HD:2 * HD]),
                              _pad16(p["e_W1"][2 * HD:]),
                              p["e_b1"].reshape(1, 2 * HD),
                              p["e_W2"], p["e_b2"].reshape(1, HD))

        agg = _scatter128(e128, row, zeros128)
        h = _node_mlp(h, agg[0], agg[1],
                      _padrows(p["n_W1"][:HD]), _padrows(p["n_W1"][HD:]),
                      p["n_b1"].reshape(1, 2 * HD),
                      jnp.pad(p["n_W2"], ((0, 0), (0, W128 - HD))),
                      jnp.pad(p["n_b2"], (0, W128 - HD)).reshape(1, W128))

    return _head(h, batch.reshape(N, 1), _padrows(params["out_W"]),
                 params["out_b"].reshape(1, 2 * HD))


# speedup vs baseline: 3.4179x; 1.1710x over previous
import functools
import jax
import jax.numpy as jnp
from jax import lax
from jax.experimental import pallas as pl
from jax.experimental.pallas import tpu as pltpu
from jax.experimental.pallas import tpu_sc as plsc

N = 10000
E = 160000
HD = 64
NG = 32
W128 = 128

NC = 2
NS = 16
NW = NC * NS
NPW = N // NS
CH = 200

EC = (83200, 76800)

BE = 3200
BN = 2000


@functools.cache
def _make_gather_pair128(Ec):
    mesh = plsc.VectorSubcoreMesh(core_axis_name="c", subcore_axis_name="s",
                                  num_cores=NC, num_subcores=NS)
    epw = Ec // NW
    NCH = epw // CH

    @functools.partial(
        pl.kernel,
        out_type=(jax.ShapeDtypeStruct((Ec, W128), jnp.float32),
                  jax.ShapeDtypeStruct((Ec, W128), jnp.float32)),
        mesh=mesh,
        scratch_types=[pltpu.VMEM((2 * CH,), jnp.int32),
                       pltpu.VMEM((2 * CH,), jnp.int32),
                       pltpu.VMEM((2 * CH, W128), jnp.float32),
                       pltpu.VMEM((2 * CH, W128), jnp.float32),
                       pltpu.SemaphoreType.DMA,
                       pltpu.SemaphoreType.DMA],
        name=f"sc_gather_pair_128_{Ec}",
    )
    def gather2(table_hbm, row_hbm, col_hbm, outr_hbm, outc_hbm,
                idx0, idx1, rows0, rows1, sem0, sem1):
        wid = lax.axis_index("s") * NC + lax.axis_index("c")
        bufs = ((idx0, rows0, sem0), (idx1, rows1, sem1))

        def start(c, b):
            idx_v, rows_v, sem = bufs[b]
            base = pl.multiple_of(wid * epw + c * CH, CH)
            pltpu.sync_copy(row_hbm.at[pl.ds(base, CH)],
                            idx_v.at[pl.ds(0, CH)])
            pltpu.sync_copy(col_hbm.at[pl.ds(base, CH)],
                            idx_v.at[pl.ds(CH, CH)])
            pltpu.async_copy(table_hbm.at[idx_v], rows_v, sem)

        def drain(c, b):
            idx_v, rows_v, sem = bufs[b]
            base = pl.multiple_of(wid * epw + c * CH, CH)
            pltpu.make_async_copy(table_hbm.at[idx_v], rows_v, sem).wait()
            pltpu.sync_copy(rows_v.at[pl.ds(0, CH)],
                            outr_hbm.at[pl.ds(base, CH)])
            pltpu.sync_copy(rows_v.at[pl.ds(CH, CH)],
                            outc_hbm.at[pl.ds(base, CH)])

        def body(i, _):
            c0 = 2 * i
            for b in range(2):
                c = c0 + b

                @pl.when(c < NCH)
                def _():
                    start(c, b)

                @pl.when((c >= 1) & (c <= NCH))
                def _():
                    drain(c - 1, 1 - b)
            return _

        lax.fori_loop(0, (NCH + 2) // 2, body, None)

    return gather2


@functools.cache
def _make_scatter128(Ec):
    mesh = plsc.VectorSubcoreMesh(core_axis_name="c", subcore_axis_name="s",
                                  num_cores=NC, num_subcores=NS)
    epw = Ec // NW
    NCH = epw // CH
    SW = 632
    SL = N - 15 * SW

    @functools.partial(
        pl.kernel,
        out_type=jax.ShapeDtypeStruct((NC, N, W128), jnp.float32),
        mesh=mesh,
        scratch_types=[pltpu.VMEM((CH,), jnp.int32),
                       pltpu.VMEM((CH, W128), jnp.float32),
                       pltpu.VMEM_SHARED((N, W128), jnp.float32),
                       pltpu.SemaphoreType.DMA],
        name=f"sc_scatter_add_128_{Ec}",
    )
    def scat(vals_hbm, idx_hbm, zeros_hbm, out_hbm,
             idx_v, vals_v, acc_sh, sem):
        cid = lax.axis_index("c")
        sid = lax.axis_index("s")
        wid = sid * NC + cid
        rbase = pl.multiple_of(sid * SW, 8)

        @pl.when(sid < 15)
        def _():
            pltpu.sync_copy(zeros_hbm.at[pl.ds(rbase, SW)],
                            acc_sh.at[pl.ds(rbase, SW)])

        @pl.when(sid == 15)
        def _():
            pltpu.sync_copy(zeros_hbm.at[pl.ds(15 * SW, SL)],
                            acc_sh.at[pl.ds(15 * SW, SL)])

        plsc.subcore_barrier()

        def body(c, _):
            base = pl.multiple_of(wid * epw + c * CH, CH)
            pltpu.sync_copy(idx_hbm.at[pl.ds(base, CH)], idx_v)
            pltpu.sync_copy(vals_hbm.at[pl.ds(base, CH)], vals_v)
            pltpu.sync_copy(vals_v, acc_sh.at[idx_v], add=True)
            return _

        lax.fori_loop(0, NCH, body, None)
        plsc.subcore_barrier()

        @pl.when(sid < 15)
        def _():
            pltpu.sync_copy(acc_sh.at[pl.ds(rbase, SW)],
                            out_hbm.at[cid].at[pl.ds(rbase, SW)])

        @pl.when(sid == 15)
        def _():
            pltpu.sync_copy(acc_sh.at[pl.ds(15 * SW, SL)],
                            out_hbm.at[cid].at[pl.ds(15 * SW, SL)])

    return scat


@functools.cache
def _make_scatter16(Ec):
    mesh = plsc.VectorSubcoreMesh(core_axis_name="c", subcore_axis_name="s",
                                  num_cores=1, num_subcores=NS)
    epw = Ec // NS
    NCH = epw // CH

    @functools.partial(
        pl.kernel,
        out_type=jax.ShapeDtypeStruct((N, 16), jnp.float32),
        mesh=mesh,
        scratch_types=[pltpu.VMEM((CH,), jnp.int32),
                       pltpu.VMEM((CH,), jnp.int32),
                       pltpu.VMEM((CH, 16), jnp.float32),
                       pltpu.VMEM((CH, 16), jnp.float32),
                       pltpu.VMEM_SHARED((N, 16), jnp.float32),
                       pltpu.SemaphoreType.DMA,
                       pltpu.SemaphoreType.DMA,
                       pltpu.SemaphoreType.DMA,
                       pltpu.SemaphoreType.DMA],
        compiler_params=pltpu.CompilerParams(use_tc_tiling_on_sc=False),
        name=f"sc_scatter_add_16_{Ec}",
    )
    def scat(vals_hbm, idx_hbm, zeros_hbm, out_hbm,
             idx0, idx1, vals0, vals1, acc_sh, seml0, seml1, sems0, sems1):
        sid = lax.axis_index("s")
        rbase = pl.multiple_of(sid * NPW, NPW)
        bufs = ((idx0, vals0, seml0, sems0), (idx1, vals1, seml1, sems1))
        pltpu.sync_copy(zeros_hbm.at[pl.ds(rbase, NPW)],
                        acc_sh.at[pl.ds(rbase, NPW)])
        plsc.subcore_barrier()

        def start(c, b):
            idx_v, vals_v, seml, _s = bufs[b]
            base = pl.multiple_of(sid * epw + c * CH, CH)
            pltpu.async_copy(idx_hbm.at[pl.ds(base, CH)], idx_v, seml)
            pltpu.async_copy(vals_hbm.at[pl.ds(base, CH)], vals_v, seml)

        def fire(c, b):
            idx_v, vals_v, seml, sems = bufs[b]
            base = pl.multiple_of(sid * epw + c * CH, CH)
            pltpu.make_async_copy(idx_hbm.at[pl.ds(base, CH)],
                                  idx_v, seml).wait()
            pltpu.make_async_copy(vals_hbm.at[pl.ds(base, CH)],
                                  vals_v, seml).wait()
            pltpu.async_copy(vals_v, acc_sh.at[idx_v], sems, add=True)

        def drain(b):
            idx_v, vals_v, _l, sems = bufs[b]
            pltpu.make_async_copy(vals_v, acc_sh.at[idx_v], sems).wait()

        def body(i, _):
            c0 = 2 * i
            for b in range(2):
                c = c0 + b

                @pl.when((c >= 2) & (c < NCH + 2))
                def _():
                    drain(b)

                @pl.when(c < NCH)
                def _():
                    start(c, b)

                @pl.when((c >= 1) & (c < NCH + 1))
                def _():
                    fire(c - 1, 1 - b)
            return _

        lax.fori_loop(0, (NCH + 3) // 2, body, None)
        plsc.subcore_barrier()
        pltpu.sync_copy(acc_sh.at[pl.ds(rbase, NPW)],
                        out_hbm.at[pl.ds(rbase, NPW)])

    return scat


@functools.cache
def _make_gather_sub16(Ec):
    mesh = plsc.VectorSubcoreMesh(core_axis_name="c", subcore_axis_name="s",
                                  num_cores=NC, num_subcores=NS)
    epw = Ec // NW
    NCH = epw // CH

    @functools.partial(
        pl.kernel,
        out_type=jax.ShapeDtypeStruct((Ec, 16), jnp.float32),
        mesh=mesh,
        scratch_types=[pltpu.VMEM((2 * CH,), jnp.int32),
                       pltpu.VMEM((2 * CH,), jnp.int32),
                       pltpu.VMEM((2 * CH, 16), jnp.float32),
                       pltpu.VMEM((2 * CH, 16), jnp.float32),
                       pltpu.SemaphoreType.DMA,
                       pltpu.SemaphoreType.DMA],
        compiler_params=pltpu.CompilerParams(use_tc_tiling_on_sc=False),
        name=f"sc_gather_sub_16_{Ec}",
    )
    def gsub(table_hbm, row_hbm, col_hbm, out_hbm,
             idx0, idx1, rows0, rows1, sem0, sem1):
        wid = lax.axis_index("s") * NC + lax.axis_index("c")
        bufs = ((idx0, rows0, sem0), (idx1, rows1, sem1))

        def start(c, b):
            idx_v, rows_v, sem = bufs[b]
            base = pl.multiple_of(wid * epw + c * CH, CH)
            pltpu.sync_copy(row_hbm.at[pl.ds(base, CH)],
                            idx_v.at[pl.ds(0, CH)])
            pltpu.sync_copy(col_hbm.at[pl.ds(base, CH)],
                            idx_v.at[pl.ds(CH, CH)])
            pltpu.async_copy(table_hbm.at[idx_v], rows_v, sem)

        def drain(c, b):
            idx_v, rows_v, sem = bufs[b]
            base = pl.multiple_of(wid * epw + c * CH, CH)
            pltpu.make_async_copy(table_hbm.at[idx_v], rows_v, sem).wait()

            def sub_body(i, _):
                rows_v[i] = rows_v[i] - rows_v[CH + i]
                return _

            lax.fori_loop(0, CH, sub_body, None)
            pltpu.sync_copy(rows_v.at[pl.ds(0, CH)],
                            out_hbm.at[pl.ds(base, CH)])

        def body(i, _):
            c0 = 2 * i
            for b in range(2):
                c = c0 + b

                @pl.when(c < NCH)
                def _():
                    start(c, b)

                @pl.when((c >= 1) & (c <= NCH))
                def _():
                    drain(c - 1, 1 - b)
            return _

        lax.fori_loop(0, (NCH + 2) // 2, body, None)

    return gsub


def _gather128(table, row, col):
    return _make_gather_pair128(row.shape[0])(table, row, col)


def _scatter128(vals, idx, zeros):
    return _make_scatter128(idx.shape[0])(vals, idx, zeros)


def _scatter16(vals, idx, zeros):
    return _make_scatter16(idx.shape[0])(vals, idx, zeros)


def _gather_sub16(table, row, col):
    return _make_gather_sub16(row.shape[0])(table, row, col)


def _eblk(i):
    return (i, 0)


def _wblk(i):
    return (0, 0)


def _wspec(shape):
    return pl.BlockSpec(shape, _wblk)


def _padrows(w):
    return jnp.pad(w, ((0, W128 - w.shape[0]), (0, 0)))


def _pad16(w):
    return jnp.pad(w, ((0, 16 - w.shape[0]), (0, 0)))


def _silu(v):
    return v * jax.nn.sigmoid(v)


def _prep_kernel(x_ref, w_ref, b_ref, o_ref):
    x0 = x_ref[...] * w_ref[...] + b_ref[...]
    o_ref[...] = jnp.concatenate(
        [x0, jnp.zeros((BN, W128 - HD), jnp.float32)], axis=-1)


def _prep(x, in_w, in_b):
    return pl.pallas_call(
        _prep_kernel,
        grid=(N // BN,),
        in_specs=[pl.BlockSpec((BN, 1), _eblk), _wspec((1, HD)),
                  _wspec((1, HD))],
        out_specs=pl.BlockSpec((BN, W128), _eblk),
        out_shape=jax.ShapeDtypeStruct((N, W128), jnp.float32),
    )(x, in_w.reshape(1, HD), in_b.reshape(1, HD))


def _coord_mlp_body(xr, xc, ra, w1a, w1b, w1c, b1, w2, b2,
                    eww, ewb, scale, delta_o):
    rel = ra[...]
    h = (jnp.dot(xr[...], w1a[...], preferred_element_type=jnp.float32)
         + jnp.dot(xc[...], w1b[...], preferred_element_type=jnp.float32)
         + jnp.dot(rel, w1c[...], preferred_element_type=jnp.float32)
         + b1[...])
    h = _silu(h)
    d = jnp.dot(h, w2[...], preferred_element_type=jnp.float32) + b2[...]
    nrm = jnp.sqrt(jnp.sum(d * d, axis=-1, keepdims=True))
    nrm = jnp.maximum(nrm, 1e-8)
    d = d / nrm * scale[...]
    ew = jax.nn.sigmoid(
        jnp.sum(rel * eww[...], axis=-1, keepdims=True) + ewb[...])
    delta_o[...] = d * ew


@functools.cache
def _make_coord_mlp(Ec):
    return pl.pallas_call(
        _coord_mlp_body,
        grid=(Ec // BE,),
        in_specs=[pl.BlockSpec((BE, W128), _eblk),
                  pl.BlockSpec((BE, W128), _eblk),
                  pl.BlockSpec((BE, 16), _eblk),
                  _wspec((W128, 2 * HD)), _wspec((W128, 2 * HD)),
                  _wspec((16, 2 * HD)), _wspec((1, 2 * HD)),
                  _wspec((2 * HD, 16)), _wspec((1, 16)),
                  _wspec((1, 16)), _wspec((1, 1)), _wspec((1, 1))],
        out_specs=pl.BlockSpec((BE, 16), _eblk),
        out_shape=jax.ShapeDtypeStruct((Ec, 16), jnp.float32),
    )


def _edge_mlp_body(xr, xc, ra, drel, w1a, w1b, w1c, b1, w2, b2, e_o, rel_o):
    rel = ra[...] + drel[...]
    h = (jnp.dot(xr[...], w1a[...], preferred_element_type=jnp.float32)
         + jnp.dot(xc[...], w1b[...], preferred_element_type=jnp.float32)
         + jnp.dot(rel, w1c[...], preferred_element_type=jnp.float32)
         + b1[...])
    h = _silu(h)
    e = _silu(
        jnp.dot(h, w2[...], preferred_element_type=jnp.float32) + b2[...])
    e_o[...] = jnp.concatenate(
        [e, jnp.zeros((BE, W128 - HD), jnp.float32)], axis=-1)
    rel_o[...] = rel


@functools.cache
def _make_edge_mlp(Ec):
    return pl.pallas_call(
        _edge_mlp_body,
        grid=(Ec // BE,),
        in_specs=[pl.BlockSpec((BE, W128), _eblk),
                  pl.BlockSpec((BE, W128), _eblk),
                  pl.BlockSpec((BE, 16), _eblk),
                  pl.BlockSpec((BE, 16), _eblk),
                  _wspec((W128, 2 * HD)), _wspec((W128, 2 * HD)),
                  _wspec((16, 2 * HD)), _wspec((1, 2 * HD)),
                  _wspec((2 * HD, HD)), _wspec((1, HD))],
        out_specs=(pl.BlockSpec((BE, W128), _eblk),
                   pl.BlockSpec((BE, 16), _eblk)),
        out_shape=(jax.ShapeDtypeStruct((Ec, W128), jnp.float32),
                   jax.ShapeDtypeStruct((Ec, 16), jnp.float32)),
    )


def _pair_add_body(a_ref, b_ref, o_ref):
    o_ref[...] = a_ref[...] + b_ref[...]


def _pair_add(a, b):
    spec = pl.BlockSpec((BN, 16), _eblk)
    return pl.pallas_call(
        _pair_add_body,
        grid=(N // BN,),
        in_specs=[spec, spec],
        out_specs=spec,
        out_shape=jax.ShapeDtypeStruct((N, 16), jnp.float32),
    )(a, b)


def _node_mlp_body(x, a0, a1, a2, a3, w1a, w1b, b1, w2, b2, o):
    xv = x[...]
    agg = (a0[...] + a1[...]) + (a2[...] + a3[...])
    h = _silu(jnp.dot(xv, w1a[...], preferred_element_type=jnp.float32)
              + jnp.dot(agg, w1b[...], preferred_element_type=jnp.float32)
              + b1[...])
    xn = xv + jnp.dot(h, w2[...], preferred_element_type=jnp.float32) + b2[...]
    mu = jnp.sum(xn, axis=-1, keepdims=True) * (1.0 / HD)
    lane = lax.broadcasted_iota(jnp.int32, (1, W128), 1)
    xc = jnp.where(lane < HD, xn - mu, 0.0)
    var = jnp.sum(xc * xc, axis=-1, keepdims=True) * (1.0 / HD)
    o[...] = xc / jnp.sqrt(var + 1e-5)


def _node_mlp(x, aggs, w1a, w1b, b1, w2, b2):
    nspec = pl.BlockSpec((BN, W128), _eblk)
    return pl.pallas_call(
        _node_mlp_body,
        grid=(N // BN,),
        in_specs=[nspec, nspec, nspec, nspec, nspec,
                  _wspec((W128, 2 * HD)), _wspec((W128, 2 * HD)),
                  _wspec((1, 2 * HD)), _wspec((2 * HD, W128)),
                  _wspec((1, W128))],
        out_specs=nspec,
        out_shape=jax.ShapeDtypeStruct((N, W128), jnp.float32),
    )(x, *aggs, w1a, w1b, b1, w2, b2)


def _head_body(x, batch, w, b, o):
    onehot = (batch[...] == lax.broadcasted_iota(jnp.int32, (1, NG), 1))
    onehot = onehot.astype(jnp.float32)
    sums = lax.dot_general(onehot, x[...], (((0,), (0,)), ((), ())),
                           preferred_element_type=jnp.float32)
    ones = jnp.ones((N, 1), jnp.float32)
    cnt = lax.dot_general(onehot, ones, (((0,), (0,)), ((), ())),
                          preferred_element_type=jnp.float32)
    pooled = sums / jnp.maximum(cnt, 1.0)
    out = (jnp.dot(jax.nn.relu(pooled), w[...],
                   preferred_element_type=jnp.float32) + b[...])
    nrm = jnp.sqrt(jnp.sum(out * out, axis=-1, keepdims=True))
    o[...] = out / jnp.maximum(nrm, 1e-12)


def _head(x, batch2d, out_w, out_b):
    return pl.pallas_call(
        _head_body,
        grid=(1,),
        in_specs=[pl.BlockSpec((N, W128), _eblk), pl.BlockSpec((N, 1), _eblk),
                  _wspec((W128, 2 * HD)), _wspec((1, 2 * HD))],
        out_specs=pl.BlockSpec((NG, 2 * HD), _eblk),
        out_shape=jax.ShapeDtypeStruct((NG, 2 * HD), jnp.float32),
    )(x, batch2d, out_w, out_b)


def kernel(x, edge_index, coords, batch, params):
    rows = []
    cols = []
    off = 0
    for Ec in EC:
        rows.append(lax.slice_in_dim(edge_index[0], off, off + Ec))
        cols.append(lax.slice_in_dim(edge_index[1], off, off + Ec))
        off += Ec
    cpad = jnp.pad(coords, ((0, 0), (0, 13)))
    zeros16 = jnp.zeros((N, 16), jnp.float32)
    zeros128 = jnp.zeros((N, W128), jnp.float32)

    h = _prep(x, params["in_W"], params["in_b"])
    rels = [_gather_sub16(cpad, r, c) for r, c in zip(rows, cols)]

    for p in params["layers"]:
        cw1a = _padrows(p["c_W1"][:HD])
        cw1b = _padrows(p["c_W1"][HD:2 * HD])
        cw1c = _pad16(p["c_W1"][2 * HD:])
        cw2 = jnp.pad(p["c_W2"], ((0, 0), (0, 13)))
        cb2 = jnp.pad(p["c_b2"], (0, 13)).reshape(1, 16)
        eww = _pad16(p["ew_W"]).reshape(1, 16)
        ew1a = _padrows(p["e_W1"][:HD])
        ew1b = _padrows(p["e_W1"][HD:2 * HD])
        ew1c = _pad16(p["e_W1"][2 * HD:])

        xs = [_gather128(h, r, c) for r, c in zip(rows, cols)]
        deltas = [
            _make_coord_mlp(Ec)(
                xs[k][0], xs[k][1], rels[k],
                cw1a, cw1b, cw1c, p["c_b1"].reshape(1, 2 * HD),
                cw2, cb2, eww, p["ew_b"].reshape(1, 1),
                p["scale"].reshape(1, 1))
            for k, Ec in enumerate(EC)]

        dcs = [_scatter16(deltas[k], rows[k], zeros16)
               for k in range(len(EC))]
        dc = _pair_add(dcs[0], dcs[1])
        drels = [_gather_sub16(dc, r, c) for r, c in zip(rows, cols)]

        aggs = []
        new_rels = []
        for k, Ec in enumerate(EC):
            e128, rel_new = _make_edge_mlp(Ec)(
                xs[k][0], xs[k][1], rels[k], drels[k],
                ew1a, ew1b, ew1c, p["e_b1"].reshape(1, 2 * HD),
                p["e_W2"], p["e_b2"].reshape(1, HD))
            new_rels.append(rel_new)
            agg2 = _scatter128(e128, rows[k], zeros128)
            aggs.extend([agg2[0], agg2[1]])
        rels = new_rels

        h = _node_mlp(h, aggs,
                      _padrows(p["n_W1"][:HD]), _padrows(p["n_W1"][HD:]),
                      p["n_b1"].reshape(1, 2 * HD),
                      jnp.pad(p["n_W2"], ((0, 0), (0, W128 - HD))),
                      jnp.pad(p["n_b2"], (0, W128 - HD)).reshape(1, W128))

    return _head(h, batch.reshape(N, 1), _padrows(params["out_W"]),
                 params["out_b"].reshape(1, 2 * HD))


# speedup vs baseline: 3.8354x; 1.1222x over previous
import functools
import jax
import jax.numpy as jnp
from jax import lax
from jax.experimental import pallas as pl
from jax.experimental.pallas import tpu as pltpu
from jax.experimental.pallas import tpu_sc as plsc

N = 10000
E = 160000
HD = 64
NG = 32
W128 = 128

NC = 2
NS = 16
NW = NC * NS
NPW = N // NS
CH = 200

EC = (83200, 76800)

BE = 3200
BN = 2000


@functools.cache
def _make_gather_pair128(Ec):
    mesh = plsc.VectorSubcoreMesh(core_axis_name="c", subcore_axis_name="s",
                                  num_cores=NC, num_subcores=NS)
    epw = Ec // NW
    NCH = epw // CH

    @functools.partial(
        pl.kernel,
        out_type=jax.ShapeDtypeStruct((Ec, W128), jnp.float32),
        mesh=mesh,
        scratch_types=[pltpu.VMEM((2 * CH,), jnp.int32),
                       pltpu.VMEM((2 * CH,), jnp.int32),
                       pltpu.VMEM((2 * CH, W128), jnp.float32),
                       pltpu.VMEM((2 * CH, W128), jnp.float32),
                       pltpu.SemaphoreType.DMA,
                       pltpu.SemaphoreType.DMA],
        name=f"sc_gather_pair_128_{Ec}",
    )
    def gather2(table_hbm, row_hbm, col_hbm, out_hbm,
                idx0, idx1, rows0, rows1, sem0, sem1):
        wid = lax.axis_index("s") * NC + lax.axis_index("c")
        bufs = ((idx0, rows0, sem0), (idx1, rows1, sem1))

        def start(c, b):
            idx_v, rows_v, sem = bufs[b]
            base = pl.multiple_of(wid * epw + c * CH, CH)
            pltpu.sync_copy(row_hbm.at[pl.ds(base, CH)],
                            idx_v.at[pl.ds(0, CH)])
            pltpu.sync_copy(col_hbm.at[pl.ds(base, CH)],
                            idx_v.at[pl.ds(CH, CH)])
            pltpu.async_copy(table_hbm.at[idx_v], rows_v, sem)

        def drain(c, b):
            idx_v, rows_v, sem = bufs[b]
            base = pl.multiple_of(wid * epw + c * CH, CH)
            pltpu.make_async_copy(table_hbm.at[idx_v], rows_v, sem).wait()
            plsc.subcore_barrier()

            def pack_body(i, _):
                for j in range(4):
                    rows_v[i, pl.ds(HD + 16 * j, 16)] = (
                        rows_v[CH + i, pl.ds(16 * j, 16)])
                return _

            lax.fori_loop(0, CH, pack_body, None)
            plsc.subcore_barrier()
            pltpu.sync_copy(rows_v.at[pl.ds(0, CH)],
                            out_hbm.at[pl.ds(base, CH)])

        def body(i, _):
            c0 = 2 * i
            for b in range(2):
                c = c0 + b

                @pl.when(c < NCH)
                def _():
                    start(c, b)

                @pl.when((c >= 1) & (c <= NCH))
                def _():
                    drain(c - 1, 1 - b)
            return _

        lax.fori_loop(0, (NCH + 2) // 2, body, None)

    return gather2


@functools.cache
def _make_scatter128(Ec):
    mesh = plsc.VectorSubcoreMesh(core_axis_name="c", subcore_axis_name="s",
                                  num_cores=NC, num_subcores=NS)
    epw = Ec // NW
    NCH = epw // CH
    SW = 632
    SL = N - 15 * SW

    @functools.partial(
        pl.kernel,
        out_type=jax.ShapeDtypeStruct((NC, N, W128), jnp.float32),
        mesh=mesh,
        scratch_types=[pltpu.VMEM((CH,), jnp.int32),
                       pltpu.VMEM((CH, W128), jnp.float32),
                       pltpu.VMEM_SHARED((N, W128), jnp.float32),
                       pltpu.SemaphoreType.DMA],
        name=f"sc_scatter_add_128_{Ec}",
    )
    def scat(vals_hbm, idx_hbm, zeros_hbm, out_hbm,
             idx_v, vals_v, acc_sh, sem):
        cid = lax.axis_index("c")
        sid = lax.axis_index("s")
        wid = sid * NC + cid
        rbase = pl.multiple_of(sid * SW, 8)

        @pl.when(sid < 15)
        def _():
            pltpu.sync_copy(zeros_hbm.at[pl.ds(rbase, SW)],
                            acc_sh.at[pl.ds(rbase, SW)])

        @pl.when(sid == 15)
        def _():
            pltpu.sync_copy(zeros_hbm.at[pl.ds(15 * SW, SL)],
                            acc_sh.at[pl.ds(15 * SW, SL)])

        plsc.subcore_barrier()

        def body(c, _):
            base = pl.multiple_of(wid * epw + c * CH, CH)
            pltpu.sync_copy(idx_hbm.at[pl.ds(base, CH)], idx_v)
            pltpu.sync_copy(vals_hbm.at[pl.ds(base, CH)], vals_v)
            pltpu.sync_copy(vals_v, acc_sh.at[idx_v], add=True)
            return _

        lax.fori_loop(0, NCH, body, None)
        plsc.subcore_barrier()

        @pl.when(sid < 15)
        def _():
            pltpu.sync_copy(acc_sh.at[pl.ds(rbase, SW)],
                            out_hbm.at[cid].at[pl.ds(rbase, SW)])

        @pl.when(sid == 15)
        def _():
            pltpu.sync_copy(acc_sh.at[pl.ds(15 * SW, SL)],
                            out_hbm.at[cid].at[pl.ds(15 * SW, SL)])

    return scat


@functools.cache
def _make_scatter16(Ec):
    mesh = plsc.VectorSubcoreMesh(core_axis_name="c", subcore_axis_name="s",
                                  num_cores=1, num_subcores=NS)
    epw = Ec // NS
    NCH = epw // CH

    @functools.partial(
        pl.kernel,
        out_type=jax.ShapeDtypeStruct((N, 16), jnp.float32),
        mesh=mesh,
        scratch_types=[pltpu.VMEM((CH,), jnp.int32),
                       pltpu.VMEM((CH,), jnp.int32),
                       pltpu.VMEM((CH, 16), jnp.float32),
                       pltpu.VMEM((CH, 16), jnp.float32),
                       pltpu.VMEM_SHARED((N, 16), jnp.float32),
                       pltpu.SemaphoreType.DMA,
                       pltpu.SemaphoreType.DMA,
                       pltpu.SemaphoreType.DMA,
                       pltpu.SemaphoreType.DMA],
        compiler_params=pltpu.CompilerParams(use_tc_tiling_on_sc=False),
        name=f"sc_scatter_add_16_{Ec}",
    )
    def scat(vals_hbm, idx_hbm, zeros_hbm, out_hbm,
             idx0, idx1, vals0, vals1, acc_sh, seml0, seml1, sems0, sems1):
        sid = lax.axis_index("s")
        rbase = pl.multiple_of(sid * NPW, NPW)
        bufs = ((idx0, vals0, seml0, sems0), (idx1, vals1, seml1, sems1))
        pltpu.sync_copy(zeros_hbm.at[pl.ds(rbase, NPW)],
                        acc_sh.at[pl.ds(rbase, NPW)])
        plsc.subcore_barrier()

        def start(c, b):
            idx_v, vals_v, seml, _s = bufs[b]
            base = pl.multiple_of(sid * epw + c * CH, CH)
            pltpu.async_copy(idx_hbm.at[pl.ds(base, CH)], idx_v, seml)
            pltpu.async_copy(vals_hbm.at[pl.ds(base, CH)], vals_v, seml)

        def fire(c, b):
            idx_v, vals_v, seml, sems = bufs[b]
            base = pl.multiple_of(sid * epw + c * CH, CH)
            pltpu.make_async_copy(idx_hbm.at[pl.ds(base, CH)],
                                  idx_v, seml).wait()
            pltpu.make_async_copy(vals_hbm.at[pl.ds(base, CH)],
                                  vals_v, seml).wait()
            pltpu.async_copy(vals_v, acc_sh.at[idx_v], sems, add=True)

        def drain(b):
            idx_v, vals_v, _l, sems = bufs[b]
            pltpu.make_async_copy(vals_v, acc_sh.at[idx_v], sems).wait()

        def body(i, _):
            c0 = 2 * i
            for b in range(2):
                c = c0 + b

                @pl.when((c >= 2) & (c < NCH + 2))
                def _():
                    drain(b)

                @pl.when(c < NCH)
                def _():
                    start(c, b)

                @pl.when((c >= 1) & (c < NCH + 1))
                def _():
                    fire(c - 1, 1 - b)
            return _

        lax.fori_loop(0, (NCH + 3) // 2, body, None)
        plsc.subcore_barrier()
        pltpu.sync_copy(acc_sh.at[pl.ds(rbase, NPW)],
                        out_hbm.at[pl.ds(rbase, NPW)])

    return scat


@functools.cache
def _make_gather_sub16(Ec):
    mesh = plsc.VectorSubcoreMesh(core_axis_name="c", subcore_axis_name="s",
                                  num_cores=NC, num_subcores=NS)
    epw = Ec // NW
    NCH = epw // CH

    @functools.partial(
        pl.kernel,
        out_type=jax.ShapeDtypeStruct((Ec, 16), jnp.float32),
        mesh=mesh,
        scratch_types=[pltpu.VMEM((2 * CH,), jnp.int32),
                       pltpu.VMEM((2 * CH,), jnp.int32),
                       pltpu.VMEM((2 * CH, 16), jnp.float32),
                       pltpu.VMEM((2 * CH, 16), jnp.float32),
                       pltpu.SemaphoreType.DMA,
                       pltpu.SemaphoreType.DMA],
        compiler_params=pltpu.CompilerParams(use_tc_tiling_on_sc=False),
        name=f"sc_gather_sub_16_{Ec}",
    )
    def gsub(table_hbm, row_hbm, col_hbm, out_hbm,
             idx0, idx1, rows0, rows1, sem0, sem1):
        wid = lax.axis_index("s") * NC + lax.axis_index("c")
        bufs = ((idx0, rows0, sem0), (idx1, rows1, sem1))

        def start(c, b):
            idx_v, rows_v, sem = bufs[b]
            base = pl.multiple_of(wid * epw + c * CH, CH)
            pltpu.sync_copy(row_hbm.at[pl.ds(base, CH)],
                            idx_v.at[pl.ds(0, CH)])
            pltpu.sync_copy(col_hbm.at[pl.ds(base, CH)],
                            idx_v.at[pl.ds(CH, CH)])
            pltpu.async_copy(table_hbm.at[idx_v], rows_v, sem)

        def drain(c, b):
            idx_v, rows_v, sem = bufs[b]
            base = pl.multiple_of(wid * epw + c * CH, CH)
            pltpu.make_async_copy(table_hbm.at[idx_v], rows_v, sem).wait()

            def sub_body(i, _):
                rows_v[i] = rows_v[i] - rows_v[CH + i]
                return _

            lax.fori_loop(0, CH, sub_body, None)
            pltpu.sync_copy(rows_v.at[pl.ds(0, CH)],
                            out_hbm.at[pl.ds(base, CH)])

        def body(i, _):
            c0 = 2 * i
            for b in range(2):
                c = c0 + b

                @pl.when(c < NCH)
                def _():
                    start(c, b)

                @pl.when((c >= 1) & (c <= NCH))
                def _():
                    drain(c - 1, 1 - b)
            return _

        lax.fori_loop(0, (NCH + 2) // 2, body, None)

    return gsub


def _gather128(table, row, col):
    return _make_gather_pair128(row.shape[0])(table, row, col)


def _scatter128(vals, idx, zeros):
    return _make_scatter128(idx.shape[0])(vals, idx, zeros)


def _scatter16(vals, idx, zeros):
    return _make_scatter16(idx.shape[0])(vals, idx, zeros)


def _gather_sub16(table, row, col):
    return _make_gather_sub16(row.shape[0])(table, row, col)


def _eblk(i):
    return (i, 0)


def _wblk(i):
    return (0, 0)


def _wspec(shape):
    return pl.BlockSpec(shape, _wblk)


def _padrows(w):
    return jnp.pad(w, ((0, W128 - w.shape[0]), (0, 0)))


def _pad16(w):
    return jnp.pad(w, ((0, 16 - w.shape[0]), (0, 0)))


def _silu(v):
    return v * jax.nn.sigmoid(v)


def _prep_kernel(x_ref, w_ref, b_ref, o_ref):
    x0 = x_ref[...] * w_ref[...] + b_ref[...]
    o_ref[...] = jnp.concatenate(
        [x0, jnp.zeros((BN, W128 - HD), jnp.float32)], axis=-1)


def _prep(x, in_w, in_b):
    return pl.pallas_call(
        _prep_kernel,
        grid=(N // BN,),
        in_specs=[pl.BlockSpec((BN, 1), _eblk), _wspec((1, HD)),
                  _wspec((1, HD))],
        out_specs=pl.BlockSpec((BN, W128), _eblk),
        out_shape=jax.ShapeDtypeStruct((N, W128), jnp.float32),
    )(x, in_w.reshape(1, HD), in_b.reshape(1, HD))


def _coord_mlp_body(xrc, ra, w1ab, w1c, b1, w2, b2,
                    eww, ewb, scale, delta_o):
    rel = ra[...]
    h = (jnp.dot(xrc[...], w1ab[...], preferred_element_type=jnp.float32)
         + jnp.dot(rel, w1c[...], preferred_element_type=jnp.float32)
         + b1[...])
    h = _silu(h)
    d = jnp.dot(h, w2[...], preferred_element_type=jnp.float32) + b2[...]
    nrm = jnp.sqrt(jnp.sum(d * d, axis=-1, keepdims=True))
    nrm = jnp.maximum(nrm, 1e-8)
    d = d / nrm * scale[...]
    ew = jax.nn.sigmoid(
        jnp.sum(rel * eww[...], axis=-1, keepdims=True) + ewb[...])
    delta_o[...] = d * ew


@functools.cache
def _make_coord_mlp(Ec):
    return pl.pallas_call(
        _coord_mlp_body,
        grid=(Ec // BE,),
        in_specs=[pl.BlockSpec((BE, W128), _eblk),
                  pl.BlockSpec((BE, 16), _eblk),
                  _wspec((W128, 2 * HD)),
                  _wspec((16, 2 * HD)), _wspec((1, 2 * HD)),
                  _wspec((2 * HD, 16)), _wspec((1, 16)),
                  _wspec((1, 16)), _wspec((1, 1)), _wspec((1, 1))],
        out_specs=pl.BlockSpec((BE, 16), _eblk),
        out_shape=jax.ShapeDtypeStruct((Ec, 16), jnp.float32),
    )


def _edge_mlp_body(xrc, ra, drel, w1ab, w1c, b1, w2, b2, e_o, rel_o):
    rel = ra[...] + drel[...]
    h = (jnp.dot(xrc[...], w1ab[...], preferred_element_type=jnp.float32)
         + jnp.dot(rel, w1c[...], preferred_element_type=jnp.float32)
         + b1[...])
    h = _silu(h)
    e = _silu(
        jnp.dot(h, w2[...], preferred_element_type=jnp.float32) + b2[...])
    e_o[...] = jnp.concatenate(
        [e, jnp.zeros((BE, W128 - HD), jnp.float32)], axis=-1)
    rel_o[...] = rel


@functools.cache
def _make_edge_mlp(Ec):
    return pl.pallas_call(
        _edge_mlp_body,
        grid=(Ec // BE,),
        in_specs=[pl.BlockSpec((BE, W128), _eblk),
                  pl.BlockSpec((BE, 16), _eblk),
                  pl.BlockSpec((BE, 16), _eblk),
                  _wspec((W128, 2 * HD)),
                  _wspec((16, 2 * HD)), _wspec((1, 2 * HD)),
                  _wspec((2 * HD, HD)), _wspec((1, HD))],
        out_specs=(pl.BlockSpec((BE, W128), _eblk),
                   pl.BlockSpec((BE, 16), _eblk)),
        out_shape=(jax.ShapeDtypeStruct((Ec, W128), jnp.float32),
                   jax.ShapeDtypeStruct((Ec, 16), jnp.float32)),
    )


def _pair_add_body(a_ref, b_ref, o_ref):
    o_ref[...] = a_ref[...] + b_ref[...]


def _pair_add(a, b):
    spec = pl.BlockSpec((BN, 16), _eblk)
    return pl.pallas_call(
        _pair_add_body,
        grid=(N // BN,),
        in_specs=[spec, spec],
        out_specs=spec,
        out_shape=jax.ShapeDtypeStruct((N, 16), jnp.float32),
    )(a, b)


def _node_mlp_body(x, a0, a1, a2, a3, w1a, w1b, b1, w2, b2, o):
    xv = x[...]
    agg = (a0[...] + a1[...]) + (a2[...] + a3[...])
    h = _silu(jnp.dot(xv, w1a[...], preferred_element_type=jnp.float32)
              + jnp.dot(agg, w1b[...], preferred_element_type=jnp.float32)
              + b1[...])
    xn = xv + jnp.dot(h, w2[...], preferred_element_type=jnp.float32) + b2[...]
    mu = jnp.sum(xn, axis=-1, keepdims=True) * (1.0 / HD)
    lane = lax.broadcasted_iota(jnp.int32, (1, W128), 1)
    xc = jnp.where(lane < HD, xn - mu, 0.0)
    var = jnp.sum(xc * xc, axis=-1, keepdims=True) * (1.0 / HD)
    o[...] = xc / jnp.sqrt(var + 1e-5)


def _node_mlp(x, aggs, w1a, w1b, b1, w2, b2):
    nspec = pl.BlockSpec((BN, W128), _eblk)
    return pl.pallas_call(
        _node_mlp_body,
        grid=(N // BN,),
        in_specs=[nspec, nspec, nspec, nspec, nspec,
                  _wspec((W128, 2 * HD)), _wspec((W128, 2 * HD)),
                  _wspec((1, 2 * HD)), _wspec((2 * HD, W128)),
                  _wspec((1, W128))],
        out_specs=nspec,
        out_shape=jax.ShapeDtypeStruct((N, W128), jnp.float32),
    )(x, *aggs, w1a, w1b, b1, w2, b2)


def _head_body(x, batch, w, b, o):
    onehot = (batch[...] == lax.broadcasted_iota(jnp.int32, (1, NG), 1))
    onehot = onehot.astype(jnp.float32)
    sums = lax.dot_general(onehot, x[...], (((0,), (0,)), ((), ())),
                           preferred_element_type=jnp.float32)
    ones = jnp.ones((N, 1), jnp.float32)
    cnt = lax.dot_general(onehot, ones, (((0,), (0,)), ((), ())),
                          preferred_element_type=jnp.float32)
    pooled = sums / jnp.maximum(cnt, 1.0)
    out = (jnp.dot(jax.nn.relu(pooled), w[...],
                   preferred_element_type=jnp.float32) + b[...])
    nrm = jnp.sqrt(jnp.sum(out * out, axis=-1, keepdims=True))
    o[...] = out / jnp.maximum(nrm, 1e-12)


def _head(x, batch2d, out_w, out_b):
    return pl.pallas_call(
        _head_body,
        grid=(1,),
        in_specs=[pl.BlockSpec((N, W128), _eblk), pl.BlockSpec((N, 1), _eblk),
                  _wspec((W128, 2 * HD)), _wspec((1, 2 * HD))],
        out_specs=pl.BlockSpec((NG, 2 * HD), _eblk),
        out_shape=jax.ShapeDtypeStruct((NG, 2 * HD), jnp.float32),
    )(x, batch2d, out_w, out_b)


def kernel(x, edge_index, coords, batch, params):
    rows = []
    cols = []
    off = 0
    for Ec in EC:
        rows.append(lax.slice_in_dim(edge_index[0], off, off + Ec))
        cols.append(lax.slice_in_dim(edge_index[1], off, off + Ec))
        off += Ec
    cpad = jnp.pad(coords, ((0, 0), (0, 13)))
    zeros16 = jnp.zeros((N, 16), jnp.float32)
    zeros128 = jnp.zeros((N, W128), jnp.float32)

    h = _prep(x, params["in_W"], params["in_b"])
    rels = [_gather_sub16(cpad, r, c) for r, c in zip(rows, cols)]

    for p in params["layers"]:
        cw1ab = p["c_W1"][:2 * HD]
        cw1c = _pad16(p["c_W1"][2 * HD:])
        cw2 = jnp.pad(p["c_W2"], ((0, 0), (0, 13)))
        cb2 = jnp.pad(p["c_b2"], (0, 13)).reshape(1, 16)
        eww = _pad16(p["ew_W"]).reshape(1, 16)
        ew1ab = p["e_W1"][:2 * HD]
        ew1c = _pad16(p["e_W1"][2 * HD:])

        xs = [_gather128(h, r, c) for r, c in zip(rows, cols)]
        deltas = [
            _make_coord_mlp(Ec)(
                xs[k], rels[k],
                cw1ab, cw1c, p["c_b1"].reshape(1, 2 * HD),
                cw2, cb2, eww, p["ew_b"].reshape(1, 1),
                p["scale"].reshape(1, 1))
            for k, Ec in enumerate(EC)]

        dcs = [_scatter16(deltas[k], rows[k], zeros16)
               for k in range(len(EC))]
        dc = _pair_add(dcs[0], dcs[1])
        drels = [_gather_sub16(dc, r, c) for r, c in zip(rows, cols)]

        aggs = []
        new_rels = []
        for k, Ec in enumerate(EC):
            e128, rel_new = _make_edge_mlp(Ec)(
                xs[k], rels[k], drels[k],
                ew1ab, ew1c, p["e_b1"].reshape(1, 2 * HD),
                p["e_W2"], p["e_b2"].reshape(1, HD))
            new_rels.append(rel_new)
            agg2 = _scatter128(e128, rows[k], zeros128)
            aggs.extend([agg2[0], agg2[1]])
        rels = new_rels

        h = _node_mlp(h, aggs,
                      _padrows(p["n_W1"][:HD]), _padrows(p["n_W1"][HD:]),
                      p["n_b1"].reshape(1, 2 * HD),
                      jnp.pad(p["n_W2"], ((0, 0), (0, W128 - HD))),
                      jnp.pad(p["n_b2"], (0, W128 - HD)).reshape(1, W128))

    return _head(h, batch.reshape(N, 1), _padrows(params["out_W"]),
                 params["out_b"].reshape(1, 2 * HD))
